# Initial kernel scaffold; baseline (speedup 1.0000x reference)
import jax
import jax.numpy as jnp
from jax.experimental import pallas as pl


def kernel(x, edge_index, W, b_dense, bias):
    raise NotImplementedError("write your pallas kernel here")


# speedup vs baseline: 4.3427x; 4.3427x over previous
import jax
import jax.numpy as jnp
from jax import lax
from jax.experimental import pallas as pl
from jax.experimental.pallas import tpu as pltpu
from jax.experimental.pallas import tpu_sc as plsc

N_NODES = 10000
D = 128
NC = 2
NS = 16
NW = NC * NS
CHUNK = 128

E = 320000
EPW = -(-E // (NW * CHUNK)) * CHUNK
E_PAD = EPW * NW
NCHUNKS = EPW // CHUNK

ZPT = 632
N_PAD = ZPT * NS
OPT = 624
OREM = N_NODES - OPT * NS


def _mm_body(x_ref, w_ref, b_ref, o_ref):
    o_ref[...] = (
        jnp.dot(x_ref[...], w_ref[...], preferred_element_type=jnp.float32)
        + b_ref[...]
    )


def _comb_body(p_ref, b_ref, o_ref):
    o_ref[...] = p_ref[0] + p_ref[1] + b_ref[...]


def _sc_body(h_hbm, src_hbm, dst_hbm, out_hbm, src_v, dst_v, rows_v, acc, sem):
    cid = lax.axis_index("c")
    sid = lax.axis_index("s")

    z16 = jnp.zeros((16,), jnp.float32)

    def _zero_row(r, carry):
        for j in range(D // 16):
            rows_v[r, pl.ds(16 * j, 16)] = z16
        return carry

    lax.fori_loop(0, CHUNK, _zero_row, 0)

    zbase = pl.multiple_of(sid * ZPT, 8)
    for k in range(ZPT // CHUNK):
        pltpu.sync_copy(rows_v.at[pl.ds(0, CHUNK)],
                        acc.at[pl.ds(zbase + k * CHUNK, CHUNK)])
    zrem = ZPT % CHUNK
    if zrem:
        pltpu.sync_copy(rows_v.at[pl.ds(0, zrem)],
                        acc.at[pl.ds(zbase + (ZPT // CHUNK) * CHUNK, zrem)])

    plsc.subcore_barrier()

    base0 = (cid * NS + sid) * EPW

    def _chunk(i, carry):
        base = pl.multiple_of(base0 + i * CHUNK, CHUNK)
        pltpu.sync_copy(src_hbm.at[pl.ds(base, CHUNK)], src_v)
        pltpu.sync_copy(dst_hbm.at[pl.ds(base, CHUNK)], dst_v)
        pltpu.async_copy(h_hbm.at[src_v], rows_v, sem).wait()
        pltpu.sync_copy(rows_v, acc.at[dst_v], add=True)
        return carry

    lax.fori_loop(0, NCHUNKS, _chunk, 0)

    plsc.subcore_barrier()

    obase = pl.multiple_of(sid * OPT, 8)
    for k in range(OPT // CHUNK):
        pltpu.sync_copy(acc.at[pl.ds(obase + k * CHUNK, CHUNK)],
                        out_hbm.at[cid].at[pl.ds(obase + k * CHUNK, CHUNK)])
    orem = OPT % CHUNK
    if orem:
        pltpu.sync_copy(acc.at[pl.ds(obase + (OPT // CHUNK) * CHUNK, orem)],
                        out_hbm.at[cid].at[pl.ds(obase + (OPT // CHUNK) * CHUNK, orem)])

    @pl.when(sid == NS - 1)
    def _tail():
        pltpu.sync_copy(acc.at[pl.ds(OPT * NS, OREM)],
                        out_hbm.at[cid].at[pl.ds(OPT * NS, OREM)])


def kernel(x, edge_index, W, b_dense, bias):
    src = edge_index[0].astype(jnp.int32)
    dst = edge_index[1].astype(jnp.int32)
    pad = E_PAD - E
    src = jnp.concatenate([src, jnp.zeros((pad,), jnp.int32)])
    dst = jnp.concatenate([dst, jnp.full((pad,), N_NODES, jnp.int32)])

    b2 = b_dense[None, :]
    h = pl.pallas_call(
        _mm_body,
        grid=(10,),
        in_specs=[
            pl.BlockSpec((N_NODES // 10, D), lambda i: (i, 0)),
            pl.BlockSpec((D, D), lambda i: (0, 0)),
            pl.BlockSpec((1, D), lambda i: (0, 0)),
        ],
        out_specs=pl.BlockSpec((N_NODES // 10, D), lambda i: (i, 0)),
        out_shape=jax.ShapeDtypeStruct((N_NODES, D), jnp.float32),
    )(x, W, b2)

    sc_fn = pl.kernel(
        _sc_body,
        out_type=jax.ShapeDtypeStruct((NC, N_NODES, D), jnp.float32),
        mesh=plsc.VectorSubcoreMesh(core_axis_name="c", subcore_axis_name="s"),
        scratch_types=[
            pltpu.VMEM((CHUNK,), jnp.int32),
            pltpu.VMEM((CHUNK,), jnp.int32),
            pltpu.VMEM((CHUNK, D), jnp.float32),
            pltpu.VMEM_SHARED((N_PAD, D), jnp.float32),
            pltpu.SemaphoreType.DMA,
        ],
    )
    partials = sc_fn(h, src, dst)

    bias2 = bias[None, :]
    out = pl.pallas_call(
        _comb_body,
        grid=(10,),
        in_specs=[
            pl.BlockSpec((NC, N_NODES // 10, D), lambda i: (0, i, 0)),
            pl.BlockSpec((1, D), lambda i: (0, 0)),
        ],
        out_specs=pl.BlockSpec((N_NODES // 10, D), lambda i: (i, 0)),
        out_shape=jax.ShapeDtypeStruct((N_NODES, D), jnp.float32),
    )(partials, bias2)
    return out
